# baseline (speedup 1.0000x reference)
import functools

import jax
import jax.numpy as jnp
from jax import lax
from jax.experimental import pallas as pl
from jax.experimental.pallas import tpu as pltpu
from jax.experimental.pallas import tpu_sc as plsc

_F = 26
_V = 100000
_H = 64
_B = 16384
_NC = 2
_NS = 16
_NW = _NC * _NS
_CB = _B // _NW
_RPC = 4
_NCH = _CB // _RPC
_K = _RPC * _F


def _make_sc_kernel():
  mesh = plsc.VectorSubcoreMesh(core_axis_name="c", subcore_axis_name="s")

  @functools.partial(
      pl.kernel,
      mesh=mesh,
      out_type=jax.ShapeDtypeStruct((_B, _H), jnp.float32),
      compiler_params=pltpu.CompilerParams(use_tc_tiling_on_sc=False),
      scratch_types=[
          pltpu.VMEM((_NCH, _K), jnp.int32),
          pltpu.VMEM((_K, _H), jnp.float32),
          pltpu.VMEM((_CB, _H), jnp.float32),
          pltpu.SemaphoreType.DMA,
      ],
  )
  def k(tab_hbm, idx_hbm, out_hbm, idx_v, buf_v, acc_v, sem):
    wid = lax.axis_index("s") * _NC + lax.axis_index("c")
    pltpu.sync_copy(idx_hbm.at[wid], idx_v)

    def chunk_body(j, carry):
      pltpu.async_copy(tab_hbm.at[idx_v.at[j]], buf_v, sem).wait()
      for r in range(_RPC):
        for g in range(_H // 16):
          v = buf_v[r * _F, pl.ds(16 * g, 16)]
          for i in range(1, _F):
            v = v + buf_v[r * _F + i, pl.ds(16 * g, 16)]
          acc_v[j * _RPC + r, pl.ds(16 * g, 16)] = v
      return carry

    lax.fori_loop(0, _NCH, chunk_body, 0)
    pltpu.sync_copy(acc_v, out_hbm.at[pl.ds(wid * _CB, _CB)])

  return k


_sc_kernel = _make_sc_kernel()


def kernel(x, tables):
  tab = tables.reshape(_F * _V, _H)
  idx = x.astype(jnp.int32) + (jnp.arange(_F, dtype=jnp.int32) * _V)[None, :]
  idx = idx.reshape(_NW, _NCH, _K)
  return _sc_kernel(tab, idx)


# speedup vs baseline: 1.5551x; 1.5551x over previous
import functools

import jax
import jax.numpy as jnp
from jax import lax
from jax.experimental import pallas as pl
from jax.experimental.pallas import tpu as pltpu
from jax.experimental.pallas import tpu_sc as plsc

_F = 26
_V = 100000
_H = 64
_B = 16384
_NC = 2
_NS = 16
_NW = _NC * _NS
_CB = _B // _NW
_RPC = 4
_NCH = _CB // _RPC
_K = _RPC * _F

_C = 4096
_NJ = -(-_V // _C)
_TR = _F * _NJ * _C


def _transpose_body(in_ref, out_ref):
  x = in_ref[0]
  xt = jnp.swapaxes(x, 0, 1)
  out_ref[...] = jnp.concatenate([xt[: _C // 2], xt[_C // 2:]], axis=1)


def _relayout(tt):
  return pl.pallas_call(
      _transpose_body,
      grid=(_F, _NJ),
      in_specs=[pl.BlockSpec((1, _H, _C), lambda i, j: (i, 0, j))],
      out_specs=pl.BlockSpec((_C // 2, 128), lambda i, j: (i * _NJ + j, 0)),
      out_shape=jax.ShapeDtypeStruct((_TR // 2, 128), jnp.float32),
  )(tt)


def _make_sc_kernel():
  mesh = plsc.VectorSubcoreMesh(core_axis_name="c", subcore_axis_name="s")

  @functools.partial(
      pl.kernel,
      mesh=mesh,
      out_type=jax.ShapeDtypeStruct((_B, _H), jnp.float32),
      compiler_params=pltpu.CompilerParams(use_tc_tiling_on_sc=False),
      scratch_types=[
          pltpu.VMEM((_NCH, _K), jnp.int32),
          pltpu.VMEM((_K, _H), jnp.float32),
          pltpu.VMEM((_CB, _H), jnp.float32),
          pltpu.SemaphoreType.DMA,
      ],
  )
  def k(tab_hbm, idx_hbm, out_hbm, idx_v, buf_v, acc_v, sem):
    wid = lax.axis_index("s") * _NC + lax.axis_index("c")
    pltpu.sync_copy(idx_hbm.at[wid], idx_v)

    def chunk_body(j, carry):
      pltpu.async_copy(tab_hbm.at[idx_v.at[j]], buf_v, sem).wait()
      for r in range(_RPC):
        for g in range(_H // 16):
          v = buf_v[r * _F, pl.ds(16 * g, 16)]
          for i in range(1, _F):
            v = v + buf_v[r * _F + i, pl.ds(16 * g, 16)]
          acc_v[j * _RPC + r, pl.ds(16 * g, 16)] = v
      return carry

    lax.fori_loop(0, _NCH, chunk_body, 0)
    pltpu.sync_copy(acc_v, out_hbm.at[pl.ds(wid * _CB, _CB)])

  return k


_sc_kernel = _make_sc_kernel()


def kernel(x, tables):
  tt = jnp.transpose(tables, (0, 2, 1))
  tab = _relayout(tt).reshape(_TR, _H)
  v = x.astype(jnp.int32)
  i_off = (jnp.arange(_F, dtype=jnp.int32) * _NJ)[None, :]
  j, q = v // _C, v % _C
  idx = ((i_off + j) * (_C // 2) + q % (_C // 2)) * 2 + q // (_C // 2)
  idx = idx.reshape(_NW, _NCH, _K)
  return _sc_kernel(tab, idx)
